# Initial kernel scaffold; baseline (speedup 1.0000x reference)
import jax
import jax.numpy as jnp
from jax.experimental import pallas as pl


def kernel(x_user, x_item, params, edge_ui, edge_iu):
    raise NotImplementedError("write your pallas kernel here")


# speedup vs baseline: 1.0170x; 1.0170x over previous
import functools

import jax
import jax.numpy as jnp
from jax.experimental import pallas as pl
from jax.experimental.pallas import tpu as pltpu


def _mm_kernel(x_ref, w_ref, b_ref, o_ref, *, act):
    y = jnp.dot(x_ref[...], w_ref[...], preferred_element_type=jnp.float32)
    y = y + b_ref[...]
    if act == "relu":
        y = jnp.maximum(y, 0.0)
    o_ref[...] = y


def _mm(x, w, b, act=None, bm=2000):
    m, k = x.shape
    n = w.shape[1]
    grid = (m // bm,)
    return pl.pallas_call(
        functools.partial(_mm_kernel, act=act),
        grid=grid,
        in_specs=[
            pl.BlockSpec((bm, k), lambda i: (i, 0)),
            pl.BlockSpec((k, n), lambda i: (0, 0)),
            pl.BlockSpec((1, n), lambda i: (0, 0)),
        ],
        out_specs=pl.BlockSpec((bm, n), lambda i: (i, 0)),
        out_shape=jax.ShapeDtypeStruct((m, n), jnp.float32),
    )(x, w, b.reshape(1, n))


def _leaky(v):
    return jnp.where(v >= 0, v, 0.2 * v)


def _elu(v):
    return jnp.where(v >= 0, v, jnp.expm1(v))


def kernel(x_user, x_item, params, edge_ui, edge_iu):
    p = params
    x_dict = {
        "user": _mm(x_user, p["W1_user"], p["b1_user"], act="relu"),
        "item": _mm(x_item, p["W1_item"], p["b1_item"], act="relu"),
    }
    edges = {("user", "item"): edge_ui, ("item", "user"): edge_iu}
    for h in range(2):
        for nt in ("user", "item"):
            x_dict[nt] = _mm(x_dict[nt], p["Wfc%d" % h], p["bfc%d" % h])
        new = {}
        for (st, tt) in (("user", "item"), ("item", "user")):
            ei = edges[(st, tt)]
            s, t = ei[0], ei[1]
            x = x_dict[st]
            hh = x_dict[tt]
            n = x.shape[0]
            a1 = p["a1_%d_%s_%s" % (h, st, tt)]
            a2 = p["a2_%d_%s_%s" % (h, st, tt)]
            x1 = x @ a1
            h1 = hh @ a2
            w2 = jnp.exp(_leaky(x1 + x @ a2))
            w1 = jnp.exp(_leaky(x1[s] + h1[t]))
            div = jax.ops.segment_sum(w1, s, num_segments=n) + w2
            agg = jax.ops.segment_sum(w1 * hh[t], s, num_segments=n) + w2 * x
            new[st] = _elu(agg / div)
        x_dict = new
    return _mm(x_dict["user"], p["Wout"], p["bout"])


# speedup vs baseline: 7.3326x; 7.2101x over previous
import functools

import jax
import jax.numpy as jnp
from jax import lax
from jax.experimental import pallas as pl
from jax.experimental.pallas import tpu as pltpu
from jax.experimental.pallas import tpu_sc as plsc

N = 10000
DH = 256
W = 144
NC, NS, L = 2, 16, 16
C = 128
E = 320000
CH_PER_SUB = -(-E // (NS * C))
E_PAD = NS * C * CH_PER_SUB
ACC_ROWS = NS * 640
RPW = 632
OUT_ROWS = NS * RPW
ZR = 64


def _mm_kernel(x_ref, w_ref, b_ref, o_ref, *, act):
    y = jnp.dot(x_ref[...], w_ref[...], preferred_element_type=jnp.float32)
    y = y + b_ref[...]
    if act == "relu":
        y = jnp.maximum(y, 0.0)
    o_ref[...] = y


def _mm(x, w, b, act=None, bm=2000):
    m, k = x.shape
    n = w.shape[1]
    return pl.pallas_call(
        functools.partial(_mm_kernel, act=act),
        grid=(m // bm,),
        in_specs=[
            pl.BlockSpec((bm, k), lambda i: (i, 0)),
            pl.BlockSpec((k, n), lambda i: (0, 0)),
            pl.BlockSpec((1, n), lambda i: (0, 0)),
        ],
        out_specs=pl.BlockSpec((bm, n), lambda i: (i, 0)),
        out_shape=jax.ShapeDtypeStruct((m, n), jnp.float32),
    )(x, w, b.reshape(1, n))


def _comb_kernel(a0_ref, a1_ref, xa1_ref, xa2_ref, x_ref, o_ref):
    v = xa1_ref[...] + xa2_ref[...]
    w2 = jnp.exp(jnp.where(v >= 0, v, 0.2 * v))
    agg = jnp.concatenate([a0_ref[...], a1_ref[...][:, : DH - W]], axis=1)
    div = a1_ref[...][:, DH - W : DH - W + 1] + w2
    y = (agg + w2 * x_ref[...]) / div
    o_ref[...] = jnp.where(y >= 0, y, jnp.exp(y) - 1.0)


def _combine(a0, a1, xa1, xa2, x, bm=2000):
    return pl.pallas_call(
        _comb_kernel,
        grid=(N // bm,),
        in_specs=[
            pl.BlockSpec((bm, W), lambda i: (i, 0)),
            pl.BlockSpec((bm, W), lambda i: (i, 0)),
            pl.BlockSpec((bm, 1), lambda i: (i, 0)),
            pl.BlockSpec((bm, 1), lambda i: (i, 0)),
            pl.BlockSpec((bm, DH), lambda i: (i, 0)),
        ],
        out_specs=pl.BlockSpec((bm, DH), lambda i: (i, 0)),
        out_shape=jax.ShapeDtypeStruct((N, DH), jnp.float32),
    )(a0, a1, xa1, xa2, x)


def _sc_pass(s_pad, tadj, x1_pad, h1_pad, table):
    mesh = plsc.VectorSubcoreMesh(
        core_axis_name="c", subcore_axis_name="s", num_cores=NC, num_subcores=NS
    )

    @functools.partial(
        pl.kernel,
        out_type=jax.ShapeDtypeStruct((NC, OUT_ROWS, W), jnp.float32),
        mesh=mesh,
        compiler_params=pltpu.CompilerParams(
            needs_layout_passes=False, use_tc_tiling_on_sc=False
        ),
        scratch_types=[
            pltpu.VMEM((10016,), jnp.float32),
            pltpu.VMEM((10016,), jnp.float32),
            pltpu.VMEM((C,), jnp.int32),
            pltpu.VMEM((C,), jnp.int32),
            pltpu.VMEM((C,), jnp.float32),
            pltpu.VMEM((C, W), jnp.float32),
            pltpu.VMEM_SHARED((ACC_ROWS, W), jnp.float32),
            pltpu.SemaphoreType.DMA,
        ],
    )
    def k(s_hbm, tadj_hbm, x1_hbm, h1_hbm, tbl_hbm, out_hbm,
          x1_v, h1_v, sidx, gidx, w_v, rows_v, acc, sem):
        c = lax.axis_index("c")
        sid = lax.axis_index("s")
        pltpu.sync_copy(x1_hbm, x1_v)
        pltpu.sync_copy(h1_hbm, h1_v)

        def zrow(r, carry):
            for cg in range(W // L):
                rows_v[r, pl.ds(cg * L, L)] = jnp.zeros((L,), jnp.float32)
            return carry

        lax.fori_loop(0, C, zrow, 0)

        def zacc(kk, carry):
            pltpu.sync_copy(rows_v, acc.at[pl.ds(sid * 640 + kk * C, C)])
            return carry

        lax.fori_loop(0, 640 // C, zacc, 0)
        plsc.subcore_barrier()

        coff = c * N
        base = sid * (CH_PER_SUB * C)

        def chunk(kk, carry):
            st = base + kk * C
            pltpu.sync_copy(s_hbm.at[pl.ds(st, C)], sidx)
            pltpu.sync_copy(tadj_hbm.at[c, pl.ds(st, C)], gidx)
            pltpu.async_copy(tbl_hbm.at[gidx], rows_v, sem).wait()
            for g in range(C // L):
                sl = pl.ds(g * L, L)
                v = plsc.load_gather(x1_v, [sidx[sl]]) + plsc.load_gather(
                    h1_v, [gidx[sl] - coff]
                )
                w_v[sl] = jnp.exp(jnp.where(v >= 0, v, 0.2 * v))

            def scale(g, cc):
                wv = w_v[pl.ds(g * L, L)]
                for j in range(L):
                    e = g * L + j
                    we = wv[j]
                    for cg in range(W // L):
                        slc = pl.ds(cg * L, L)
                        rows_v[e, slc] = rows_v[e, slc] * we
                return cc

            lax.fori_loop(0, C // L, scale, 0)
            pltpu.sync_copy(rows_v, acc.at[sidx], add=True)
            return carry

        lax.fori_loop(0, CH_PER_SUB, chunk, 0)
        plsc.subcore_barrier()
        pltpu.sync_copy(
            acc.at[pl.ds(sid * RPW, RPW)], out_hbm.at[c, pl.ds(sid * RPW, RPW)]
        )

    return k(s_pad, tadj, x1_pad, h1_pad, table)


def _mk_table(x):
    ones = jnp.ones((N, 1), jnp.float32)
    zpad = jnp.zeros((N, W - (DH - W) - 1), jnp.float32)
    hi = jnp.concatenate([x[:, W:DH], ones, zpad], axis=1)
    return jnp.concatenate([x[:, :W], hi], axis=0)


def kernel(x_user, x_item, params, edge_ui, edge_iu):
    p = params
    f32 = jnp.float32
    xu = _mm(x_user, p["W1_user"], p["b1_user"], act="relu")
    xi = _mm(x_item, p["W1_item"], p["b1_item"], act="relu")

    pads = jnp.full((E_PAD - E,), N, jnp.int32)
    padt = jnp.full((E_PAD - E,), N, jnp.int32)

    def prep(ei):
        s = ei[0].astype(jnp.int32)
        t = ei[1].astype(jnp.int32)
        s_pad = jnp.concatenate([s, pads])
        tadj = jnp.stack(
            [jnp.concatenate([t, padt]), jnp.concatenate([t + N, padt])]
        )
        return s_pad, tadj

    s_ui, tadj_ui = prep(edge_ui)
    s_iu, tadj_iu = prep(edge_iu)
    z3 = jnp.zeros((3,), f32)

    for h in range(2):
        xu = _mm(xu, p["Wfc%d" % h], p["bfc%d" % h])
        xi = _mm(xi, p["Wfc%d" % h], p["bfc%d" % h])
        au = jnp.concatenate(
            [p["a1_%d_user_item" % h], p["a2_%d_user_item" % h],
             p["a2_%d_item_user" % h]], axis=1)
        ai = jnp.concatenate(
            [p["a2_%d_user_item" % h], p["a1_%d_item_user" % h],
             p["a2_%d_item_user" % h]], axis=1)
        pu = _mm(xu, au, z3)
        pi = _mm(xi, ai, z3)

        x1u = jnp.pad(pu[:, 0], (0, 16))
        h1i = jnp.pad(pi[:, 0], (0, 16))
        x1i = jnp.pad(pi[:, 1], (0, 16))
        h1u = jnp.pad(pu[:, 2], (0, 16))

        out_ui = _sc_pass(s_ui, tadj_ui, x1u, h1i, _mk_table(xi))
        out_iu = _sc_pass(s_iu, tadj_iu, x1i, h1u, _mk_table(xu))

        xu = _combine(out_ui[0, :N], out_ui[1, :N], pu[:, 0:1], pu[:, 1:2], xu)
        xi = _combine(out_iu[0, :N], out_iu[1, :N], pi[:, 1:2], pi[:, 2:3], xi)

    return _mm(xu, p["Wout"], p["bout"])
